# Initial kernel scaffold; baseline (speedup 1.0000x reference)
import jax
import jax.numpy as jnp
from jax.experimental import pallas as pl


def kernel(loc_data, conf_data, priors, targets):
    raise NotImplementedError("write your pallas kernel here")


# speedup vs baseline: 17.5958x; 17.5958x over previous
import jax
import jax.numpy as jnp
from jax import lax
from jax.experimental import pallas as pl
from jax.experimental.pallas import tpu as pltpu

_B, _P, _G = 16, 20000, 50
_ROWS, _LANES = 160, 128
_PPAD = _ROWS * _LANES
_NEGPOS = 3
_THRESH = 0.5
_V0, _V1 = 0.1, 0.2


def _body(tgt_ref, loc_ref, conf_ref, pri_ref, out_ref, bp_ref):
    b = pl.program_id(0)
    f32 = jnp.float32
    riota = lax.broadcasted_iota(jnp.int32, (_ROWS, _LANES), 0)
    ciota = lax.broadcasted_iota(jnp.int32, (_ROWS, _LANES), 1)
    fiota = riota * _LANES + ciota

    pcx = pri_ref[0]
    pcy = pri_ref[1]
    pw = pri_ref[2]
    ph = pri_ref[3]
    px1 = pcx - pw / 2
    py1 = pcy - ph / 2
    px2 = pcx + pw / 2
    py2 = pcy + ph / 2
    parea = (px2 - px1) * (py2 - py1)

    def tgt(g, c):
        return tgt_ref[0, 0, g * 5 + c]

    def iou_row(g):
        gx1, gy1, gx2, gy2 = tgt(g, 0), tgt(g, 1), tgt(g, 2), tgt(g, 3)
        ga = (gx2 - gx1) * (gy2 - gy1)
        iw = jnp.maximum(jnp.minimum(px2, gx2) - jnp.maximum(px1, gx1), 0.0)
        ih = jnp.maximum(jnp.minimum(py2, gy2) - jnp.maximum(py1, gy1), 0.0)
        inter = iw * ih
        return inter / (ga + parea - inter)

    def g_body(g, carry):
        bov, bidx = carry
        iou = iou_row(g)
        m = jnp.max(iou)
        bp_ref[g] = jnp.min(jnp.where(iou == m, fiota, jnp.int32(2**30)))
        upd = iou > bov
        return jnp.where(upd, iou, bov), jnp.where(upd, g, bidx)

    bov, bidx = lax.fori_loop(
        0, _G, g_body,
        (jnp.full((_ROWS, _LANES), -1.0, f32),
         jnp.zeros((_ROWS, _LANES), jnp.int32)))

    def f_body(g, carry):
        bov, bidx = carry
        mk = fiota == bp_ref[g]
        return jnp.where(mk, 2.0, bov), jnp.where(mk, g, bidx)

    bov, bidx = lax.fori_loop(0, _G, f_body, (bov, bidx))

    def m_body(g, carry):
        mx1, my1, mx2, my2 = carry
        mk = bidx == g
        return (jnp.where(mk, tgt(g, 0), mx1), jnp.where(mk, tgt(g, 1), my1),
                jnp.where(mk, tgt(g, 2), mx2), jnp.where(mk, tgt(g, 3), my2))

    z = jnp.zeros((_ROWS, _LANES), f32)
    mx1, my1, mx2, my2 = lax.fori_loop(0, _G, m_body, (z, z, z, z))

    pos = bov >= _THRESH

    gcx = ((mx1 + mx2) / 2 - pcx) / (_V0 * pw)
    gcy = ((my1 + my2) / 2 - pcy) / (_V0 * ph)
    gw = jnp.log((mx2 - mx1) / pw) / _V1
    gh = jnp.log((my2 - my1) / ph) / _V1
    l0 = loc_ref[0, 0]
    l1 = loc_ref[0, 1]
    l2 = loc_ref[0, 2]

    def sl1(d):
        ad = jnp.abs(d)
        return jnp.where(ad < 1.0, 0.5 * d * d, ad - 0.5)

    sl = sl1(l0 - gcx) + sl1(l1 - gcy) + sl1(l2 - gw) + sl1(l2 - gh)
    loss_l_b = jnp.sum(jnp.where(pos, sl, 0.0))

    c0 = conf_ref[0, 0]
    c1 = conf_ref[0, 1]
    mm = jnp.maximum(c0, c1)
    lse = jnp.log(jnp.exp(c0 - mm) + jnp.exp(c1 - mm)) + mm
    ce = lse - jnp.where(pos, c1, c0)
    npos_b = jnp.sum(jnp.where(pos, 1, 0).astype(jnp.int32))
    ce_pos_b = jnp.sum(jnp.where(pos, ce, 0.0))
    valid = fiota < _P
    neg_pool = jnp.where(pos | (~valid), 0.0, ce)

    k = jnp.minimum(_NEGPOS * npos_b, _P - 1)
    bits = lax.bitcast_convert_type(neg_pool, jnp.int32)

    def bs_body(i, lo):
        cand = lo + (jnp.int32(1) << (jnp.int32(30) - i))
        cnt = jnp.sum(jnp.where(bits >= cand, 1, 0).astype(jnp.int32))
        return jnp.where(cnt >= k, cand, lo)

    lo = lax.fori_loop(0, 31, bs_body, jnp.int32(0))
    t = lax.bitcast_convert_type(lo, f32)
    gtm = bits > lo
    sum_gt = jnp.sum(jnp.where(gtm, neg_pool, 0.0))
    cnt_gt = jnp.sum(jnp.where(gtm, 1, 0).astype(jnp.int32))
    topk_b = sum_gt + (k - cnt_gt).astype(f32) * t

    r8 = lax.broadcasted_iota(jnp.int32, (8, 128), 0)
    c8 = lax.broadcasted_iota(jnp.int32, (8, 128), 1)
    contrib = (jnp.where((r8 == 0) & (c8 == 0), loss_l_b, 0.0)
               + jnp.where((r8 == 0) & (c8 == 1), ce_pos_b + topk_b, 0.0)
               + jnp.where((r8 == 0) & (c8 == 2), npos_b.astype(f32), 0.0))

    @pl.when(b == 0)
    def _():
        out_ref[...] = jnp.zeros((8, 128), f32)

    out_ref[...] = out_ref[...] + contrib


def kernel(loc_data, conf_data, priors, targets):
    pad = _PPAD - _P
    locp = jnp.pad(loc_data, ((0, 0), (0, pad), (0, 0)))
    locp = locp.transpose(0, 2, 1).reshape(_B, 3, _ROWS, _LANES)
    confp = jnp.pad(conf_data, ((0, 0), (0, pad), (0, 0)))
    confp = confp.transpose(0, 2, 1).reshape(_B, 2, _ROWS, _LANES)
    dummy = jnp.tile(jnp.array([[2.0, 2.0, 0.1, 0.1]], jnp.float32), (pad, 1))
    prip = jnp.concatenate([priors, dummy], axis=0).T.reshape(4, _ROWS, _LANES)
    tgt = targets.reshape(_B, 1, _G * 5)

    out = pl.pallas_call(
        _body,
        grid=(_B,),
        in_specs=[
            pl.BlockSpec((1, 1, _G * 5), lambda b: (b, 0, 0),
                         memory_space=pltpu.SMEM),
            pl.BlockSpec((1, 3, _ROWS, _LANES), lambda b: (b, 0, 0, 0)),
            pl.BlockSpec((1, 2, _ROWS, _LANES), lambda b: (b, 0, 0, 0)),
            pl.BlockSpec((4, _ROWS, _LANES), lambda b: (0, 0, 0)),
        ],
        out_specs=pl.BlockSpec((8, 128), lambda b: (0, 0)),
        out_shape=jax.ShapeDtypeStruct((8, 128), jnp.float32),
        scratch_shapes=[pltpu.SMEM((_G,), jnp.int32)],
    )(tgt, locp, confp, prip)

    n = out[0, 2]
    return (out[0, 0] / n, out[0, 1] / n)


# speedup vs baseline: 28.1744x; 1.6012x over previous
import jax
import jax.numpy as jnp
from jax import lax
from jax.experimental import pallas as pl
from jax.experimental.pallas import tpu as pltpu

_B, _P, _G = 16, 20000, 50
_ROWS, _LANES = 160, 128
_PPAD = _ROWS * _LANES
_NEGPOS = 3
_THRESH = 0.5
_V0, _V1 = 0.1, 0.2


def _body(tgt_ref, loc_ref, conf_ref, pri_ref, out_ref, bp_ref):
    b = pl.program_id(0)
    f32 = jnp.float32
    riota = lax.broadcasted_iota(jnp.int32, (_ROWS, _LANES), 0)
    ciota = lax.broadcasted_iota(jnp.int32, (_ROWS, _LANES), 1)
    fiota = riota * _LANES + ciota

    pcx = pri_ref[0]
    pcy = pri_ref[1]
    pw = pri_ref[2]
    ph = pri_ref[3]
    px1 = pcx - pw / 2
    py1 = pcy - ph / 2
    px2 = pcx + pw / 2
    py2 = pcy + ph / 2
    parea = (px2 - px1) * (py2 - py1)

    def tgt(g, c):
        return tgt_ref[0, 0, g * 5 + c]

    def iou_row(g):
        gx1, gy1, gx2, gy2 = tgt(g, 0), tgt(g, 1), tgt(g, 2), tgt(g, 3)
        ga = (gx2 - gx1) * (gy2 - gy1)
        iw = jnp.maximum(jnp.minimum(px2, gx2) - jnp.maximum(px1, gx1), 0.0)
        ih = jnp.maximum(jnp.minimum(py2, gy2) - jnp.maximum(py1, gy1), 0.0)
        inter = iw * ih
        return inter / (ga + parea - inter)

    bov = jnp.full((_ROWS, _LANES), -1.0, f32)
    bidx = jnp.zeros((_ROWS, _LANES), jnp.int32)
    for g in range(_G):
        iou = iou_row(g)
        m = jnp.max(iou)
        bp_ref[g] = jnp.min(jnp.where(iou == m, fiota, jnp.int32(2**30)))
        upd = iou > bov
        bov = jnp.where(upd, iou, bov)
        bidx = jnp.where(upd, g, bidx)

    for g in range(_G):
        mk = fiota == bp_ref[g]
        bov = jnp.where(mk, 2.0, bov)
        bidx = jnp.where(mk, g, bidx)

    z = jnp.zeros((_ROWS, _LANES), f32)
    mx1, my1, mx2, my2 = z, z, z, z
    for g in range(_G):
        mk = bidx == g
        mx1 = jnp.where(mk, tgt(g, 0), mx1)
        my1 = jnp.where(mk, tgt(g, 1), my1)
        mx2 = jnp.where(mk, tgt(g, 2), mx2)
        my2 = jnp.where(mk, tgt(g, 3), my2)

    pos = bov >= _THRESH

    gcx = ((mx1 + mx2) / 2 - pcx) / (_V0 * pw)
    gcy = ((my1 + my2) / 2 - pcy) / (_V0 * ph)
    gw = jnp.log((mx2 - mx1) / pw) / _V1
    gh = jnp.log((my2 - my1) / ph) / _V1
    l0 = loc_ref[0, 0]
    l1 = loc_ref[0, 1]
    l2 = loc_ref[0, 2]

    def sl1(d):
        ad = jnp.abs(d)
        return jnp.where(ad < 1.0, 0.5 * d * d, ad - 0.5)

    sl = sl1(l0 - gcx) + sl1(l1 - gcy) + sl1(l2 - gw) + sl1(l2 - gh)
    loss_l_b = jnp.sum(jnp.where(pos, sl, 0.0))

    c0 = conf_ref[0, 0]
    c1 = conf_ref[0, 1]
    mm = jnp.maximum(c0, c1)
    lse = jnp.log(jnp.exp(c0 - mm) + jnp.exp(c1 - mm)) + mm
    ce = lse - jnp.where(pos, c1, c0)
    npos_b = jnp.sum(jnp.where(pos, 1, 0).astype(jnp.int32))
    ce_pos_b = jnp.sum(jnp.where(pos, ce, 0.0))
    valid = fiota < _P
    neg_pool = jnp.where(pos | (~valid), 0.0, ce)

    k = jnp.minimum(_NEGPOS * npos_b, _P - 1)
    bits = lax.bitcast_convert_type(neg_pool, jnp.int32)

    def bs_body(i, lo):
        cand = lo + (jnp.int32(1) << (jnp.int32(30) - i))
        cnt = jnp.sum(jnp.where(bits >= cand, 1, 0).astype(jnp.int32))
        return jnp.where(cnt >= k, cand, lo)

    lo = lax.fori_loop(0, 31, bs_body, jnp.int32(0))
    t = lax.bitcast_convert_type(lo, f32)
    gtm = bits > lo
    sum_gt = jnp.sum(jnp.where(gtm, neg_pool, 0.0))
    cnt_gt = jnp.sum(jnp.where(gtm, 1, 0).astype(jnp.int32))
    topk_b = sum_gt + (k - cnt_gt).astype(f32) * t

    r8 = lax.broadcasted_iota(jnp.int32, (8, 128), 0)
    c8 = lax.broadcasted_iota(jnp.int32, (8, 128), 1)
    contrib = (jnp.where((r8 == 0) & (c8 == 0), loss_l_b, 0.0)
               + jnp.where((r8 == 0) & (c8 == 1), ce_pos_b + topk_b, 0.0)
               + jnp.where((r8 == 0) & (c8 == 2), npos_b.astype(f32), 0.0))

    @pl.when(b == 0)
    def _():
        out_ref[...] = jnp.zeros((8, 128), f32)

    out_ref[...] = out_ref[...] + contrib


def kernel(loc_data, conf_data, priors, targets):
    pad = _PPAD - _P
    locp = jnp.pad(loc_data, ((0, 0), (0, pad), (0, 0)))
    locp = locp.transpose(0, 2, 1).reshape(_B, 3, _ROWS, _LANES)
    confp = jnp.pad(conf_data, ((0, 0), (0, pad), (0, 0)))
    confp = confp.transpose(0, 2, 1).reshape(_B, 2, _ROWS, _LANES)
    dummy = jnp.tile(jnp.array([[2.0, 2.0, 0.1, 0.1]], jnp.float32), (pad, 1))
    prip = jnp.concatenate([priors, dummy], axis=0).T.reshape(4, _ROWS, _LANES)
    tgt = targets.reshape(_B, 1, _G * 5)

    out = pl.pallas_call(
        _body,
        grid=(_B,),
        in_specs=[
            pl.BlockSpec((1, 1, _G * 5), lambda b: (b, 0, 0),
                         memory_space=pltpu.SMEM),
            pl.BlockSpec((1, 3, _ROWS, _LANES), lambda b: (b, 0, 0, 0)),
            pl.BlockSpec((1, 2, _ROWS, _LANES), lambda b: (b, 0, 0, 0)),
            pl.BlockSpec((4, _ROWS, _LANES), lambda b: (0, 0, 0)),
        ],
        out_specs=pl.BlockSpec((8, 128), lambda b: (0, 0)),
        out_shape=jax.ShapeDtypeStruct((8, 128), jnp.float32),
        scratch_shapes=[pltpu.SMEM((_G,), jnp.int32)],
    )(tgt, locp, confp, prip)

    n = out[0, 2]
    return (out[0, 0] / n, out[0, 1] / n)


# speedup vs baseline: 30.0144x; 1.0653x over previous
import functools

import jax
import jax.numpy as jnp
from jax import lax
from jax.experimental import pallas as pl
from jax.experimental.pallas import tpu as pltpu
from jax.experimental.pallas import tpu_sc as plsc

_B, _P, _G = 16, 20000, 50
_ROWS, _LANES = 160, 128
_PPAD = _ROWS * _LANES
_NEGPOS = 3
_THRESH = 0.5
_V0, _V1 = 0.1, 0.2


def _body(tgt_ref, loc_ref, conf_ref, pri_ref, out_ref, pool_ref, bp_ref):
    b = pl.program_id(0)
    f32 = jnp.float32
    riota = lax.broadcasted_iota(jnp.int32, (_ROWS, _LANES), 0)
    ciota = lax.broadcasted_iota(jnp.int32, (_ROWS, _LANES), 1)
    fiota = riota * _LANES + ciota

    pcx = pri_ref[0]
    pcy = pri_ref[1]
    pw = pri_ref[2]
    ph = pri_ref[3]
    px1 = pcx - pw / 2
    py1 = pcy - ph / 2
    px2 = pcx + pw / 2
    py2 = pcy + ph / 2
    parea = (px2 - px1) * (py2 - py1)

    def tgt(g, c):
        return tgt_ref[0, 0, g * 5 + c]

    def iou_row(g):
        gx1, gy1, gx2, gy2 = tgt(g, 0), tgt(g, 1), tgt(g, 2), tgt(g, 3)
        ga = (gx2 - gx1) * (gy2 - gy1)
        iw = jnp.maximum(jnp.minimum(px2, gx2) - jnp.maximum(px1, gx1), 0.0)
        ih = jnp.maximum(jnp.minimum(py2, gy2) - jnp.maximum(py1, gy1), 0.0)
        inter = iw * ih
        return inter / (ga + parea - inter)

    bov = jnp.full((_ROWS, _LANES), -1.0, f32)
    bidx = jnp.zeros((_ROWS, _LANES), jnp.int32)
    for g in range(_G):
        iou = iou_row(g)
        m = jnp.max(iou)
        bp_ref[g] = jnp.min(jnp.where(iou == m, fiota, jnp.int32(2**30)))
        upd = iou > bov
        bov = jnp.where(upd, iou, bov)
        bidx = jnp.where(upd, g, bidx)

    for g in range(_G):
        mk = fiota == bp_ref[g]
        bov = jnp.where(mk, 2.0, bov)
        bidx = jnp.where(mk, g, bidx)

    z = jnp.zeros((_ROWS, _LANES), f32)
    mx1, my1, mx2, my2 = z, z, z, z
    for g in range(_G):
        mk = bidx == g
        mx1 = jnp.where(mk, tgt(g, 0), mx1)
        my1 = jnp.where(mk, tgt(g, 1), my1)
        mx2 = jnp.where(mk, tgt(g, 2), mx2)
        my2 = jnp.where(mk, tgt(g, 3), my2)

    pos = bov >= _THRESH

    gcx = ((mx1 + mx2) / 2 - pcx) / (_V0 * pw)
    gcy = ((my1 + my2) / 2 - pcy) / (_V0 * ph)
    gw = jnp.log((mx2 - mx1) / pw) / _V1
    gh = jnp.log((my2 - my1) / ph) / _V1
    l0 = loc_ref[0, 0]
    l1 = loc_ref[0, 1]
    l2 = loc_ref[0, 2]

    def sl1(d):
        ad = jnp.abs(d)
        return jnp.where(ad < 1.0, 0.5 * d * d, ad - 0.5)

    sl = sl1(l0 - gcx) + sl1(l1 - gcy) + sl1(l2 - gw) + sl1(l2 - gh)
    loss_l_b = jnp.sum(jnp.where(pos, sl, 0.0))

    c0 = conf_ref[0, 0]
    c1 = conf_ref[0, 1]
    mm = jnp.maximum(c0, c1)
    lse = jnp.log(jnp.exp(c0 - mm) + jnp.exp(c1 - mm)) + mm
    ce = lse - jnp.where(pos, c1, c0)
    npos_b = jnp.sum(jnp.where(pos, 1, 0).astype(jnp.int32))
    ce_pos_b = jnp.sum(jnp.where(pos, ce, 0.0))
    valid = fiota < _P
    pool_ref[...] = jnp.where(pos | (~valid), 0.0, ce).reshape(1, _ROWS, _LANES)

    r8 = lax.broadcasted_iota(jnp.int32, (8, 128), 0)
    c8 = lax.broadcasted_iota(jnp.int32, (8, 128), 1)
    contrib = (jnp.where((r8 == 0) & (c8 == 0), loss_l_b, 0.0)
               + jnp.where((r8 == 0) & (c8 == 1), ce_pos_b, 0.0)
               + jnp.where((r8 == 0) & (c8 == 2), npos_b.astype(f32), 0.0)
               + jnp.where((r8 == 1) & (c8 == b), npos_b.astype(f32), 0.0))

    @pl.when(b == 0)
    def _():
        out_ref[...] = jnp.zeros((8, 128), f32)

    out_ref[...] = out_ref[...] + contrib


_CHUNKS = _PPAD // 16


def _topk_sc_body(pool_f_hbm, pool_i_hbm, k_hbm, out_hbm, lo_hbm,
                  buf_f, buf_i, kv, red, outv, lov):
    wid = lax.axis_index("s") * 2 + lax.axis_index("c")

    @pl.when(wid < _B)
    def _():
        pltpu.sync_copy(pool_f_hbm.at[wid], buf_f)
        pltpu.sync_copy(pool_i_hbm.at[wid], buf_i)
        pltpu.sync_copy(k_hbm.at[wid], kv)
        kvec = kv[...]

        def csum16(x):
            for sh in (8, 4, 2, 1):
                red[pl.ds(0, 16)] = x
                red[pl.ds(16, 16)] = x
                x = x + red[pl.ds(sh, 16)]
            return x

        def count_ge(cand):
            def cbody(j, acc):
                v = buf_i[pl.ds(j * 16, 16)]
                return acc + jnp.where(v >= cand, 1, 0)
            acc = lax.fori_loop(0, _CHUNKS, cbody,
                                jnp.zeros((16,), jnp.int32), unroll=8)
            return csum16(acc)

        def bs(i, lo):
            cand = lo + (jnp.int32(1) << (jnp.int32(30) - i))
            return jnp.where(count_ge(cand) >= kvec, cand, lo)

        lo = lax.fori_loop(0, 31, bs, jnp.zeros((16,), jnp.int32))

        def fbody(j, carry):
            fs, cnt = carry
            m = buf_i[pl.ds(j * 16, 16)] > lo
            fs = fs + jnp.where(m, buf_f[pl.ds(j * 16, 16)], 0.0)
            cnt = cnt + jnp.where(m, 1.0, 0.0)
            return fs, cnt

        fs, cnt = lax.fori_loop(
            0, _CHUNKS, fbody,
            (jnp.zeros((16,), jnp.float32), jnp.zeros((16,), jnp.float32)),
            unroll=8)
        outv[0] = fs
        outv[1] = cnt
        lov[...] = lo
        pltpu.sync_copy(outv, out_hbm.at[wid])
        pltpu.sync_copy(lov, lo_hbm.at[wid])


@functools.cache
def _make_topk_sc():
    return functools.partial(
        pl.kernel,
        mesh=plsc.VectorSubcoreMesh(core_axis_name="c", subcore_axis_name="s"),
        out_type=[
            jax.ShapeDtypeStruct((_B, 2, 16), jnp.float32),
            jax.ShapeDtypeStruct((_B, 16), jnp.int32),
        ],
        scratch_types=[
            pltpu.VMEM((_PPAD,), jnp.float32),
            pltpu.VMEM((_PPAD,), jnp.int32),
            pltpu.VMEM((16,), jnp.int32),
            pltpu.VMEM((32,), jnp.int32),
            pltpu.VMEM((2, 16), jnp.float32),
            pltpu.VMEM((16,), jnp.int32),
        ],
    )(_topk_sc_body)


def kernel(loc_data, conf_data, priors, targets):
    pad = _PPAD - _P
    locp = jnp.pad(loc_data, ((0, 0), (0, pad), (0, 0)))
    locp = locp.transpose(0, 2, 1).reshape(_B, 3, _ROWS, _LANES)
    confp = jnp.pad(conf_data, ((0, 0), (0, pad), (0, 0)))
    confp = confp.transpose(0, 2, 1).reshape(_B, 2, _ROWS, _LANES)
    dummy = jnp.tile(jnp.array([[2.0, 2.0, 0.1, 0.1]], jnp.float32), (pad, 1))
    prip = jnp.concatenate([priors, dummy], axis=0).T.reshape(4, _ROWS, _LANES)
    tgt = targets.reshape(_B, 1, _G * 5)

    out, pool = pl.pallas_call(
        _body,
        grid=(_B,),
        in_specs=[
            pl.BlockSpec((1, 1, _G * 5), lambda b: (b, 0, 0),
                         memory_space=pltpu.SMEM),
            pl.BlockSpec((1, 3, _ROWS, _LANES), lambda b: (b, 0, 0, 0)),
            pl.BlockSpec((1, 2, _ROWS, _LANES), lambda b: (b, 0, 0, 0)),
            pl.BlockSpec((4, _ROWS, _LANES), lambda b: (0, 0, 0)),
        ],
        out_specs=[
            pl.BlockSpec((8, 128), lambda b: (0, 0)),
            pl.BlockSpec((1, _ROWS, _LANES), lambda b: (b, 0, 0)),
        ],
        out_shape=[
            jax.ShapeDtypeStruct((8, 128), jnp.float32),
            jax.ShapeDtypeStruct((_B, _ROWS, _LANES), jnp.float32),
        ],
        scratch_shapes=[pltpu.SMEM((_G,), jnp.int32)],
    )(tgt, locp, confp, prip)

    pool_f = pool.reshape(_B, _PPAD)
    pool_i = lax.bitcast_convert_type(pool_f, jnp.int32)
    npos_vec = out[1, :_B].astype(jnp.int32)
    kvec = jnp.minimum(_NEGPOS * npos_vec, _P - 1)
    kmat = jnp.broadcast_to(kvec[:, None], (_B, 16))
    res, lo = _make_topk_sc()(pool_f, pool_i, kmat)
    sum_gt = jnp.sum(res[:, 0, :], axis=1)
    cnt_gt = jnp.sum(res[:, 1, :], axis=1)
    t = lax.bitcast_convert_type(lo[:, 0], jnp.float32)
    topk = sum_gt + (kvec.astype(jnp.float32) - cnt_gt) * t

    n = out[0, 2]
    return (out[0, 0] / n, (out[0, 1] + jnp.sum(topk)) / n)
